# Initial kernel scaffold; baseline (speedup 1.0000x reference)
import jax
import jax.numpy as jnp
from jax.experimental import pallas as pl


def kernel(x, edge_index, W1, a_src1, a_dst1, b1, W2, a_src2, a_dst2, b2, Wlin, blin):
    raise NotImplementedError("write your pallas kernel here")


# speedup vs baseline: 1.0552x; 1.0552x over previous
import functools

import jax
import jax.numpy as jnp
from jax.experimental import pallas as pl
from jax.experimental.pallas import tpu as pltpu

N = 10000
E = 160000
F_IN = 256
NHID = 256
H1 = 8
NCLS = 64

_MP = 10240
_RB = 512


def _mm_kernel(x_ref, w_ref, o_ref):
    o_ref[...] = jnp.dot(x_ref[...], w_ref[...],
                         preferred_element_type=jnp.float32)


def _mm(x, w):
    MP, K = x.shape
    _, C = w.shape
    return pl.pallas_call(
        _mm_kernel,
        grid=(MP // _RB,),
        in_specs=[
            pl.BlockSpec((_RB, K), lambda i: (i, 0)),
            pl.BlockSpec((K, C), lambda i: (0, 0)),
        ],
        out_specs=pl.BlockSpec((_RB, C), lambda i: (i, 0)),
        out_shape=jax.ShapeDtypeStruct((MP, C), jnp.float32),
    )(x, w)


def _gat_layer(h_pad, src, dst, W, a_src, a_dst, bias, heads, out_ch):
    h = _mm(h_pad, W)[:N]
    hr = h.reshape(N, heads, out_ch)
    alpha_src = (hr * a_src[None]).sum(-1)
    alpha_dst = (hr * a_dst[None]).sum(-1)
    e = alpha_src[src] + alpha_dst[dst]
    e = jnp.where(e >= 0, e, 0.2 * e)
    m = jnp.max(e, axis=0)
    p = jnp.exp(e - m[None])
    denom = jax.ops.segment_sum(p, dst, num_segments=N)
    alpha = p / (denom[dst] + 1e-16)
    msg = hr[src] * alpha[:, :, None]
    out = jax.ops.segment_sum(msg, dst, num_segments=N)
    return out.reshape(N, heads * out_ch) + bias


def kernel(x, edge_index, W1, a_src1, a_dst1, b1, W2, a_src2, a_dst2, b2,
           Wlin, blin):
    loop = jnp.arange(N, dtype=edge_index.dtype)
    src = jnp.concatenate([edge_index[0], loop])
    dst = jnp.concatenate([edge_index[1], loop])

    xp = jnp.zeros((_MP, F_IN), x.dtype).at[:N].set(x)
    h = _gat_layer(xp, src, dst, W1, a_src1, a_dst1, b1, H1, NHID)
    h = jnp.tanh(h)
    hp = jnp.zeros((_MP, H1 * NHID), h.dtype).at[:N].set(h)
    h = _gat_layer(hp, src, dst, W2, a_src2, a_dst2, b2, 1, NHID)
    h = jnp.tanh(h)
    hp = jnp.zeros((_MP, NHID), h.dtype).at[:N].set(h)
    logits = _mm(hp, Wlin)[:N] + blin
    return jax.nn.log_softmax(logits, axis=1)
